# baseline (speedup 1.0000x reference)
import functools

import jax
import jax.numpy as jnp
import numpy as np
from jax import lax
from jax.experimental import pallas as pl
from jax.experimental.pallas import tpu as pltpu
from jax.experimental.pallas import tpu_sc as plsc

_PRIMES = [99991, 100003, 100019, 100043, 100057, 100069, 100103, 100109]
_EMBED_DIM = 16

_NC = 2
_NS = 16
_NW = _NC * _NS
_LANES = 16

_BQ = 1024
_GROW = 128
_NG = _BQ // _GROW
_NBUF = 2


def _offsets_np():
    offs = [0]
    for p in _PRIMES[:-1]:
        offs.append(offs[-1] + p)
    return np.asarray(offs, dtype=np.int32)


@functools.partial(jax.jit, static_argnames=("units_per_w",))
def _sc_gather(table, idx6, off16, units_per_w):
    s_dim, c_dim, h_dim, l_dim = idx6.shape
    n_units = s_dim * h_dim * (c_dim * l_dim // _BQ)
    mesh = plsc.VectorSubcoreMesh(core_axis_name="c", subcore_axis_name="s")

    @functools.partial(
        pl.kernel,
        mesh=mesh,
        out_type=jax.ShapeDtypeStruct(
            (s_dim, h_dim, 2, c_dim, 8, l_dim), jnp.float32),
        compiler_params=pltpu.CompilerParams(
            use_tc_tiling_on_sc=False, needs_layout_passes=False),
        scratch_types=[
            pltpu.VMEM((_NBUF, _NG, _GROW), jnp.int32),
            pltpu.VMEM((_NBUF, _BQ, _EMBED_DIM), jnp.float32),
            pltpu.VMEM((_NBUF, 2, _NG, 8, _GROW), jnp.float32),
            pltpu.VMEM((8, _LANES), jnp.int32),
            pltpu.SemaphoreType.DMA,
            pltpu.SemaphoreType.DMA,
            pltpu.SemaphoreType.DMA,
            pltpu.SemaphoreType.DMA,
            pltpu.SemaphoreType.DMA,
            pltpu.SemaphoreType.DMA,
        ],
    )
    def body(table_hbm, idx_hbm, off_hbm, out_hbm, idx_v, rows_v, trans_v,
             off_v, si0, si1, sg0, sg1, so0, so1):
        sem_i = (si0, si1)
        sem_g = (sg0, sg1)
        sem_o = (so0, so1)
        wid = lax.axis_index("s") * _NC + lax.axis_index("c")
        base_u = wid * units_per_w
        pltpu.sync_copy(off_hbm, off_v)
        iota16 = lax.iota(jnp.int32, _LANES)

        def decode(u):
            pair = u >> 2
            q = u & 3
            return pair >> 3, pair & 7, q * _NG

        def fire_idx(slot, u):
            s, h, c0 = decode(u)
            pltpu.async_copy(idx_hbm.at[s, pl.ds(c0, _NG), h],
                             idx_v.at[slot], sem_i[slot])

        def wait_idx(slot):
            pltpu.make_async_copy(idx_hbm.at[0, pl.ds(0, _NG), 0],
                                  idx_v.at[slot], sem_i[slot]).wait()

        def do_adds(slot, u):
            _, h, _ = decode(u)
            off_b = off_v[h, :]
            for j in range(_NG):
                for k in range(_GROW // _LANES):
                    sl = pl.ds(k * _LANES, _LANES)
                    idx_v[slot, j, sl] = idx_v[slot, j, sl] + off_b

        def fire_gathers(slot):
            for j in range(_NG):
                pltpu.async_copy(
                    table_hbm.at[idx_v.at[slot, j]],
                    rows_v.at[slot, pl.ds(j * _GROW, _GROW)],
                    sem_g[slot],
                )

        def wait_gathers(slot):
            pltpu.make_async_copy(table_hbm.at[pl.ds(0, _BQ)],
                                  rows_v.at[slot], sem_g[slot]).wait()

        cols = [jnp.full((_LANES,), d, jnp.int32) for d in range(_EMBED_DIM)]

        def do_transpose(slot):
            rows = rows_v.at[slot]

            def tbody(c_p, _):
                base = c_p * _GROW
                for l0 in range(0, _GROW, _LANES):
                    row_idx = base + (l0 + iota16)
                    for d in range(_EMBED_DIM):
                        v = plsc.load_gather(rows, [row_idx, cols[d]])
                        trans_v[slot, d // 8, c_p, d % 8,
                                pl.ds(l0, _LANES)] = v
                return _

            lax.fori_loop(0, _NG, tbody, 0)

        def fire_out(slot, u):
            s, h, c0 = decode(u)
            for r2 in range(2):
                pltpu.async_copy(
                    trans_v.at[slot, r2],
                    out_hbm.at[s, h, r2, pl.ds(c0, _NG)],
                    sem_o[slot],
                )

        def wait_out(slot):
            for r2 in range(2):
                pltpu.make_async_copy(trans_v.at[slot, r2],
                                      out_hbm.at[0, 0, 0, pl.ds(0, _NG)],
                                      sem_o[slot]).wait()

        for b in range(_NBUF):
            fire_idx(b, base_u + b)
        for b in range(_NBUF):
            wait_idx(b)
            do_adds(b, base_u + b)
            fire_gathers(b)
        for b in range(_NBUF):
            wait_gathers(b)
            do_transpose(b)
            fire_out(b, base_u + b)
            fire_idx(b, jnp.minimum(base_u + _NBUF + b, n_units - 1))

        def loop_body(g, _):
            u0 = base_u + _NBUF * g
            for b in range(_NBUF):
                wait_idx(b)
                do_adds(b, u0 + b)
                fire_gathers(b)
            for b in range(_NBUF):
                wait_gathers(b)
                wait_out(b)
                do_transpose(b)
                fire_out(b, u0 + b)
                fire_idx(b, jnp.minimum(u0 + _NBUF + b, n_units - 1))
            return _

        lax.fori_loop(1, units_per_w // _NBUF, loop_body, 0)

        for b in range(_NBUF):
            wait_idx(b)
            wait_out(b)

    return body(table, idx6, off16)


def kernel(table, hash_indices):
    bb, s_dim, h_dim = hash_indices.shape
    idx6 = (hash_indices.astype(jnp.int32)
            .transpose(1, 2, 0)
            .reshape(s_dim, h_dim, bb // 128, 128)
            .transpose(0, 2, 1, 3))
    off16 = jnp.asarray(
        np.repeat(_offsets_np()[:, None], _LANES, axis=1), dtype=jnp.int32)
    n_units = s_dim * h_dim * (bb // _BQ)
    out6 = _sc_gather(table, idx6, off16, n_units // _NW)
    return (out6.transpose(3, 5, 0, 1, 2, 4)
            .reshape(bb, s_dim, h_dim, _EMBED_DIM))


# speedup vs baseline: 1.5250x; 1.5250x over previous
import functools

import jax
import jax.numpy as jnp
import numpy as np
from jax import lax
from jax.experimental import pallas as pl
from jax.experimental.pallas import tpu as pltpu
from jax.experimental.pallas import tpu_sc as plsc

_PRIMES = [99991, 100003, 100019, 100043, 100057, 100069, 100103, 100109]
_EMBED_DIM = 16

_NC = 2
_NS = 16
_NW = _NC * _NS
_LANES = 16

_BQ = 1024
_GROW = 128
_NG = _BQ // _GROW
_NBUF = 2


def _offsets_np():
    offs = [0]
    for p in _PRIMES[:-1]:
        offs.append(offs[-1] + p)
    return np.asarray(offs, dtype=np.int32)


@functools.partial(jax.jit, static_argnames=("units_per_w",))
def _sc_gather(table, idx6, off16, units_per_w):
    s_dim, c_dim, h_dim, l_dim = idx6.shape
    n_units = s_dim * h_dim * (c_dim * l_dim // _BQ)
    mesh = plsc.VectorSubcoreMesh(core_axis_name="c", subcore_axis_name="s")

    @functools.partial(
        pl.kernel,
        mesh=mesh,
        out_type=jax.ShapeDtypeStruct(
            (s_dim, h_dim, 2, c_dim, 8, l_dim), jnp.float32),
        compiler_params=pltpu.CompilerParams(
            use_tc_tiling_on_sc=False, needs_layout_passes=False),
        scratch_types=[
            pltpu.VMEM((_NBUF, _NG, _GROW), jnp.int32),
            pltpu.VMEM((_NBUF, _BQ, _EMBED_DIM), jnp.float32),
            pltpu.VMEM((_NBUF, 2, _NG, 8, _GROW), jnp.float32),
            pltpu.VMEM((8, _LANES), jnp.int32),
            pltpu.SemaphoreType.DMA,
            pltpu.SemaphoreType.DMA,
            pltpu.SemaphoreType.DMA,
            pltpu.SemaphoreType.DMA,
            pltpu.SemaphoreType.DMA,
            pltpu.SemaphoreType.DMA,
        ],
    )
    def body(table_hbm, idx_hbm, off_hbm, out_hbm, idx_v, rows_v, trans_v,
             off_v, si0, si1, sg0, sg1, so0, so1):
        sem_i = (si0, si1)
        sem_g = (sg0, sg1)
        sem_o = (so0, so1)
        wid = lax.axis_index("s") * _NC + lax.axis_index("c")
        base_u = wid * units_per_w
        pltpu.sync_copy(off_hbm, off_v)
        iota16 = lax.iota(jnp.int32, _LANES)

        def decode(u):
            pair = u >> 2
            q = u & 3
            return pair >> 3, pair & 7, q * _NG

        def fire_idx(slot, u):
            s, h, c0 = decode(u)
            pltpu.async_copy(idx_hbm.at[s, pl.ds(c0, _NG), h],
                             idx_v.at[slot], sem_i[slot])

        def wait_idx(slot):
            pltpu.make_async_copy(idx_hbm.at[0, pl.ds(0, _NG), 0],
                                  idx_v.at[slot], sem_i[slot]).wait()

        def do_adds(slot, u):
            _, h, _ = decode(u)
            off_b = off_v[h, :]
            for j in range(_NG):
                for k in range(_GROW // _LANES):
                    sl = pl.ds(k * _LANES, _LANES)
                    idx_v[slot, j, sl] = idx_v[slot, j, sl] + off_b

        def fire_gathers(slot):
            for j in range(_NG):
                pltpu.async_copy(
                    table_hbm.at[idx_v.at[slot, j]],
                    rows_v.at[slot, pl.ds(j * _GROW, _GROW)],
                    sem_g[slot],
                )

        def wait_gathers(slot):
            pltpu.make_async_copy(table_hbm.at[pl.ds(0, _BQ)],
                                  rows_v.at[slot], sem_g[slot]).wait()

        cols = [jnp.full((_LANES,), d, jnp.int32) for d in range(_EMBED_DIM)]

        def do_transpose(slot):
            rows = rows_v.at[slot]

            def tbody(c_p, _):
                base = c_p * _GROW
                for l0 in range(0, _GROW, _LANES):
                    row_idx = base + (l0 + iota16)
                    vs = [plsc.load_gather(rows, [row_idx, cols[d]])
                          for d in range(_EMBED_DIM)]
                    for d in range(_EMBED_DIM):
                        trans_v[slot, d // 8, c_p, d % 8,
                                pl.ds(l0, _LANES)] = vs[d]
                return _

            lax.fori_loop(0, _NG, tbody, 0)

        def fire_out(slot, u):
            s, h, c0 = decode(u)
            for r2 in range(2):
                pltpu.async_copy(
                    trans_v.at[slot, r2],
                    out_hbm.at[s, h, r2, pl.ds(c0, _NG)],
                    sem_o[slot],
                )

        def wait_out(slot):
            for r2 in range(2):
                pltpu.make_async_copy(trans_v.at[slot, r2],
                                      out_hbm.at[0, 0, 0, pl.ds(0, _NG)],
                                      sem_o[slot]).wait()

        for b in range(_NBUF):
            fire_idx(b, base_u + b)
        for b in range(_NBUF):
            wait_idx(b)
            do_adds(b, base_u + b)
            fire_gathers(b)
        for b in range(_NBUF):
            wait_gathers(b)
            do_transpose(b)
            fire_out(b, base_u + b)
            fire_idx(b, jnp.minimum(base_u + _NBUF + b, n_units - 1))

        def loop_body(g, _):
            u0 = base_u + _NBUF * g
            for b in range(_NBUF):
                wait_idx(b)
                do_adds(b, u0 + b)
                fire_gathers(b)
            for b in range(_NBUF):
                wait_gathers(b)
                wait_out(b)
                do_transpose(b)
                fire_out(b, u0 + b)
                fire_idx(b, jnp.minimum(u0 + _NBUF + b, n_units - 1))
            return _

        lax.fori_loop(1, units_per_w // _NBUF, loop_body, 0)

        for b in range(_NBUF):
            wait_idx(b)
            wait_out(b)

    return body(table, idx6, off16)


def kernel(table, hash_indices):
    bb, s_dim, h_dim = hash_indices.shape
    idx6 = (hash_indices.astype(jnp.int32)
            .transpose(1, 2, 0)
            .reshape(s_dim, h_dim, bb // 128, 128)
            .transpose(0, 2, 1, 3))
    off16 = jnp.asarray(
        np.repeat(_offsets_np()[:, None], _LANES, axis=1), dtype=jnp.int32)
    n_units = s_dim * h_dim * (bb // _BQ)
    out6 = _sc_gather(table, idx6, off16, n_units // _NW)
    return (out6.transpose(3, 5, 0, 1, 2, 4)
            .reshape(bb, s_dim, h_dim, _EMBED_DIM))


# speedup vs baseline: 2.7891x; 1.8290x over previous
import functools

import jax
import jax.numpy as jnp
import numpy as np
from jax import lax
from jax.experimental import pallas as pl
from jax.experimental.pallas import tpu as pltpu
from jax.experimental.pallas import tpu_sc as plsc

_PRIMES = [99991, 100003, 100019, 100043, 100057, 100069, 100103, 100109]
_EMBED_DIM = 16

_NC = 2
_NS = 16
_NW = _NC * _NS
_LANES = 16

_BQ = 1024
_GROW = 128
_NG = _BQ // _GROW
_NBUF = 2


def _offsets_np():
    offs = [0]
    for p in _PRIMES[:-1]:
        offs.append(offs[-1] + p)
    return np.asarray(offs, dtype=np.int32)


_VPAD = 851968
_TCH = 1024


@jax.jit
def _sc_table_rowmajor(tp4):
    mesh = plsc.VectorSubcoreMesh(core_axis_name="c", subcore_axis_name="s")
    cpw = _VPAD // (_NW * _TCH)
    ccols = _TCH // 128

    @functools.partial(
        pl.kernel,
        mesh=mesh,
        out_type=jax.ShapeDtypeStruct((_VPAD, _EMBED_DIM), jnp.float32),
        compiler_params=pltpu.CompilerParams(
            use_tc_tiling_on_sc=False, needs_layout_passes=False),
        scratch_types=[
            pltpu.VMEM((_NBUF, 2, _TCH // 128, 8, 128), jnp.float32),
            pltpu.VMEM((_NBUF, _TCH, _EMBED_DIM), jnp.float32),
            pltpu.SemaphoreType.DMA,
            pltpu.SemaphoreType.DMA,
            pltpu.SemaphoreType.DMA,
            pltpu.SemaphoreType.DMA,
        ],
    )
    def body(tp4_hbm, tlin_hbm, din, dout, si0, si1, so0, so1):
        sem_i = (si0, si1)
        sem_o = (so0, so1)
        wid = lax.axis_index("s") * _NC + lax.axis_index("c")
        base_c = wid * cpw
        iota16 = lax.iota(jnp.int32, _LANES)
        cols = [jnp.full((_LANES,), d, jnp.int32) for d in range(_EMBED_DIM)]

        def fire_in(slot, c):
            pltpu.async_copy(tp4_hbm.at[:, pl.ds(c * ccols, ccols)],
                             din.at[slot], sem_i[slot])

        def wait_in(slot):
            pltpu.make_async_copy(tp4_hbm.at[:, pl.ds(0, ccols)],
                                  din.at[slot], sem_i[slot]).wait()

        def fire_out(slot, c):
            pltpu.async_copy(dout.at[slot],
                             tlin_hbm.at[pl.ds(c * _TCH, _TCH)], sem_o[slot])

        def wait_out(slot):
            pltpu.make_async_copy(dout.at[slot],
                                  tlin_hbm.at[pl.ds(0, _TCH)],
                                  sem_o[slot]).wait()

        def do_transpose(slot):
            dref = dout.at[slot]

            def g_body(g, _):
                c_rel = g >> 3
                l0 = (g & 7) * _LANES
                row_idx = c_rel * 128 + l0 + iota16
                vs = [din[slot, d // 8, c_rel, d % 8, pl.ds(l0, _LANES)]
                      for d in range(_EMBED_DIM)]
                for d in range(_EMBED_DIM):
                    plsc.store_scatter(dref, [row_idx, cols[d]], vs[d])
                return _

            lax.fori_loop(0, _TCH // _LANES, g_body, 0)

        for b in range(_NBUF):
            fire_in(b, base_c + b)
        for b in range(_NBUF):
            wait_in(b)
            do_transpose(b)
            fire_out(b, base_c + b)
            fire_in(b, jnp.minimum(base_c + _NBUF + b, _VPAD // _TCH - 1))

        def loop_body(g, _):
            c0 = base_c + _NBUF * g
            for b in range(_NBUF):
                wait_in(b)
                wait_out(b)
                do_transpose(b)
                fire_out(b, c0 + b)
                fire_in(b, jnp.minimum(c0 + _NBUF + b, _VPAD // _TCH - 1))
            return _

        lax.fori_loop(1, cpw // _NBUF, loop_body, 0)

        for b in range(_NBUF):
            wait_in(b)
            wait_out(b)

    return body(tp4)


@functools.partial(jax.jit, static_argnames=("units_per_w",))
def _sc_gather(table, idx6, off16, units_per_w):
    s_dim, c_dim, h_dim, l_dim = idx6.shape
    n_units = s_dim * h_dim * (c_dim * l_dim // _BQ)
    mesh = plsc.VectorSubcoreMesh(core_axis_name="c", subcore_axis_name="s")

    @functools.partial(
        pl.kernel,
        mesh=mesh,
        out_type=jax.ShapeDtypeStruct(
            (s_dim, h_dim, 2, c_dim, 8, l_dim), jnp.float32),
        compiler_params=pltpu.CompilerParams(
            use_tc_tiling_on_sc=False, needs_layout_passes=False),
        scratch_types=[
            pltpu.VMEM((_NBUF, _NG, _GROW), jnp.int32),
            pltpu.VMEM((_NBUF, _BQ, _EMBED_DIM), jnp.float32),
            pltpu.VMEM((_NBUF, 2, _NG, 8, _GROW), jnp.float32),
            pltpu.VMEM((8, _LANES), jnp.int32),
            pltpu.SemaphoreType.DMA,
            pltpu.SemaphoreType.DMA,
            pltpu.SemaphoreType.DMA,
            pltpu.SemaphoreType.DMA,
            pltpu.SemaphoreType.DMA,
            pltpu.SemaphoreType.DMA,
        ],
    )
    def body(table_hbm, idx_hbm, off_hbm, out_hbm, idx_v, rows_v, trans_v,
             off_v, si0, si1, sg0, sg1, so0, so1):
        sem_i = (si0, si1)
        sem_g = (sg0, sg1)
        sem_o = (so0, so1)
        wid = lax.axis_index("s") * _NC + lax.axis_index("c")
        base_u = wid * units_per_w
        pltpu.sync_copy(off_hbm, off_v)
        iota16 = lax.iota(jnp.int32, _LANES)

        def decode(u):
            pair = u >> 2
            q = u & 3
            return pair >> 3, pair & 7, q * _NG

        def fire_idx(slot, u):
            s, h, c0 = decode(u)
            pltpu.async_copy(idx_hbm.at[s, pl.ds(c0, _NG), h],
                             idx_v.at[slot], sem_i[slot])

        def wait_idx(slot):
            pltpu.make_async_copy(idx_hbm.at[0, pl.ds(0, _NG), 0],
                                  idx_v.at[slot], sem_i[slot]).wait()

        def do_adds(slot, u):
            _, h, _ = decode(u)
            off_b = off_v[h, :]
            for j in range(_NG):
                for k in range(_GROW // _LANES):
                    sl = pl.ds(k * _LANES, _LANES)
                    idx_v[slot, j, sl] = idx_v[slot, j, sl] + off_b

        def fire_gathers(slot):
            for j in range(_NG):
                pltpu.async_copy(
                    table_hbm.at[idx_v.at[slot, j]],
                    rows_v.at[slot, pl.ds(j * _GROW, _GROW)],
                    sem_g[slot],
                )

        def wait_gathers(slot):
            pltpu.make_async_copy(table_hbm.at[pl.ds(0, _BQ)],
                                  rows_v.at[slot], sem_g[slot]).wait()

        cols = [jnp.full((_LANES,), d, jnp.int32) for d in range(_EMBED_DIM)]

        def do_transpose(slot):
            rows = rows_v.at[slot]

            def tbody(c_p, _):
                base = c_p * _GROW
                for l0 in range(0, _GROW, _LANES):
                    row_idx = base + (l0 + iota16)
                    vs = [plsc.load_gather(rows, [row_idx, cols[d]])
                          for d in range(_EMBED_DIM)]
                    for d in range(_EMBED_DIM):
                        trans_v[slot, d // 8, c_p, d % 8,
                                pl.ds(l0, _LANES)] = vs[d]
                return _

            lax.fori_loop(0, _NG, tbody, 0)

        def fire_out(slot, u):
            s, h, c0 = decode(u)
            for r2 in range(2):
                pltpu.async_copy(
                    trans_v.at[slot, r2],
                    out_hbm.at[s, h, r2, pl.ds(c0, _NG)],
                    sem_o[slot],
                )

        def wait_out(slot):
            for r2 in range(2):
                pltpu.make_async_copy(trans_v.at[slot, r2],
                                      out_hbm.at[0, 0, 0, pl.ds(0, _NG)],
                                      sem_o[slot]).wait()

        for b in range(_NBUF):
            fire_idx(b, base_u + b)
        for b in range(_NBUF):
            wait_idx(b)
            do_adds(b, base_u + b)
            fire_gathers(b)
        for b in range(_NBUF):
            wait_gathers(b)
            do_transpose(b)
            fire_out(b, base_u + b)
            fire_idx(b, jnp.minimum(base_u + _NBUF + b, n_units - 1))

        def loop_body(g, _):
            u0 = base_u + _NBUF * g
            for b in range(_NBUF):
                wait_idx(b)
                do_adds(b, u0 + b)
                fire_gathers(b)
            for b in range(_NBUF):
                wait_gathers(b)
                wait_out(b)
                do_transpose(b)
                fire_out(b, u0 + b)
                fire_idx(b, jnp.minimum(u0 + _NBUF + b, n_units - 1))
            return _

        lax.fori_loop(1, units_per_w // _NBUF, loop_body, 0)

        for b in range(_NBUF):
            wait_idx(b)
            wait_out(b)

    return body(table, idx6, off16)


def kernel(table, hash_indices):
    bb, s_dim, h_dim = hash_indices.shape
    idx6 = (hash_indices.astype(jnp.int32)
            .transpose(1, 2, 0)
            .reshape(s_dim, h_dim, bb // 128, 128)
            .transpose(0, 2, 1, 3))
    off16 = jnp.asarray(
        np.repeat(_offsets_np()[:, None], _LANES, axis=1), dtype=jnp.int32)
    tp4 = (jnp.pad(table.T, ((0, 0), (0, _VPAD - table.shape[0])))
           .reshape(2, 8, _VPAD // 128, 128)
           .transpose(0, 2, 1, 3))
    tlin = _sc_table_rowmajor(tp4)
    n_units = s_dim * h_dim * (bb // _BQ)
    out6 = _sc_gather(tlin, idx6, off16, n_units // _NW)
    return (out6.transpose(3, 5, 0, 1, 2, 4)
            .reshape(bb, s_dim, h_dim, _EMBED_DIM))


# speedup vs baseline: 2.8001x; 1.0039x over previous
import functools

import jax
import jax.numpy as jnp
import numpy as np
from jax import lax
from jax.experimental import pallas as pl
from jax.experimental.pallas import tpu as pltpu
from jax.experimental.pallas import tpu_sc as plsc

_PRIMES = [99991, 100003, 100019, 100043, 100057, 100069, 100103, 100109]
_EMBED_DIM = 16

_NC = 2
_NS = 16
_NW = _NC * _NS
_LANES = 16

_BQ = 1024
_GROW = 128
_NG = _BQ // _GROW
_NBUF = 2


def _offsets_np():
    offs = [0]
    for p in _PRIMES[:-1]:
        offs.append(offs[-1] + p)
    return np.asarray(offs, dtype=np.int32)


_VPAD = 851968
_TCH = 1024


@jax.jit
def _sc_table_rowmajor(tp4):
    mesh = plsc.VectorSubcoreMesh(core_axis_name="c", subcore_axis_name="s")
    cpw = _VPAD // (_NW * _TCH)
    ccols = _TCH // 128

    @functools.partial(
        pl.kernel,
        mesh=mesh,
        out_type=jax.ShapeDtypeStruct((_VPAD, _EMBED_DIM), jnp.float32),
        compiler_params=pltpu.CompilerParams(
            use_tc_tiling_on_sc=False, needs_layout_passes=False),
        scratch_types=[
            pltpu.VMEM((_NBUF, 2, _TCH // 128, 8, 128), jnp.float32),
            pltpu.VMEM((_NBUF, _TCH, _EMBED_DIM), jnp.float32),
            pltpu.SemaphoreType.DMA,
            pltpu.SemaphoreType.DMA,
            pltpu.SemaphoreType.DMA,
            pltpu.SemaphoreType.DMA,
        ],
    )
    def body(tp4_hbm, tlin_hbm, din, dout, si0, si1, so0, so1):
        sem_i = (si0, si1)
        sem_o = (so0, so1)
        wid = lax.axis_index("s") * _NC + lax.axis_index("c")
        base_c = wid * cpw
        iota16 = lax.iota(jnp.int32, _LANES)
        cols = [jnp.full((_LANES,), d, jnp.int32) for d in range(_EMBED_DIM)]

        def fire_in(slot, c):
            pltpu.async_copy(tp4_hbm.at[:, pl.ds(c * ccols, ccols)],
                             din.at[slot], sem_i[slot])

        def wait_in(slot):
            pltpu.make_async_copy(tp4_hbm.at[:, pl.ds(0, ccols)],
                                  din.at[slot], sem_i[slot]).wait()

        def fire_out(slot, c):
            pltpu.async_copy(dout.at[slot],
                             tlin_hbm.at[pl.ds(c * _TCH, _TCH)], sem_o[slot])

        def wait_out(slot):
            pltpu.make_async_copy(dout.at[slot],
                                  tlin_hbm.at[pl.ds(0, _TCH)],
                                  sem_o[slot]).wait()

        def do_transpose(slot):
            dref = dout.at[slot]

            def g_body(g, _):
                c_rel = g >> 3
                l0 = (g & 7) * _LANES
                row_idx = c_rel * 128 + l0 + iota16
                vs = [din[slot, d // 8, c_rel, d % 8, pl.ds(l0, _LANES)]
                      for d in range(_EMBED_DIM)]
                for d in range(_EMBED_DIM):
                    plsc.store_scatter(dref, [row_idx, cols[d]], vs[d])
                return _

            lax.fori_loop(0, _TCH // _LANES, g_body, 0)

        for b in range(_NBUF):
            fire_in(b, base_c + b)
        for b in range(_NBUF):
            wait_in(b)
            do_transpose(b)
            fire_out(b, base_c + b)
            fire_in(b, jnp.minimum(base_c + _NBUF + b, _VPAD // _TCH - 1))

        def loop_body(g, _):
            c0 = base_c + _NBUF * g
            for b in range(_NBUF):
                wait_in(b)
                wait_out(b)
                do_transpose(b)
                fire_out(b, c0 + b)
                fire_in(b, jnp.minimum(c0 + _NBUF + b, _VPAD // _TCH - 1))
            return _

        lax.fori_loop(1, cpw // _NBUF, loop_body, 0)

        for b in range(_NBUF):
            wait_in(b)
            wait_out(b)

    return body(tp4)


@functools.partial(jax.jit, static_argnames=("units_per_w",))
def _sc_gather(table, idx6, off16, units_per_w):
    s_dim, c_dim, h_dim, l_dim = idx6.shape
    n_units = s_dim * h_dim * (c_dim * l_dim // _BQ)
    mesh = plsc.VectorSubcoreMesh(core_axis_name="c", subcore_axis_name="s")

    @functools.partial(
        pl.kernel,
        mesh=mesh,
        out_type=jax.ShapeDtypeStruct(
            (s_dim, h_dim, 2, c_dim, 8, l_dim), jnp.float32),
        compiler_params=pltpu.CompilerParams(
            use_tc_tiling_on_sc=False, needs_layout_passes=False),
        scratch_types=[
            pltpu.VMEM((_NBUF, _NG, _GROW), jnp.int32),
            pltpu.VMEM((_NBUF, _BQ, _EMBED_DIM), jnp.float32),
            pltpu.VMEM((_NBUF, 2, _NG, 8, _GROW), jnp.float32),
            pltpu.VMEM((8, _LANES), jnp.int32),
            pltpu.SemaphoreType.DMA,
            pltpu.SemaphoreType.DMA,
            pltpu.SemaphoreType.DMA,
            pltpu.SemaphoreType.DMA,
            pltpu.SemaphoreType.DMA,
            pltpu.SemaphoreType.DMA,
        ],
    )
    def body(table_hbm, idx_hbm, off_hbm, out_hbm, idx_v, rows_v, trans_v,
             off_v, si0, si1, sg0, sg1, so0, so1):
        sem_i = (si0, si1)
        sem_g = (sg0, sg1)
        sem_o = (so0, so1)
        wid = lax.axis_index("s") * _NC + lax.axis_index("c")
        base_u = wid * units_per_w
        pltpu.sync_copy(off_hbm, off_v)
        iota16 = lax.iota(jnp.int32, _LANES)

        def decode(u):
            pair = u >> 2
            q = u & 3
            return pair >> 3, pair & 7, q * _NG

        def fire_idx(slot, u):
            s, h, c0 = decode(u)
            pltpu.async_copy(idx_hbm.at[s, pl.ds(c0, _NG), h],
                             idx_v.at[slot], sem_i[slot])

        def wait_idx(slot):
            pltpu.make_async_copy(idx_hbm.at[0, pl.ds(0, _NG), 0],
                                  idx_v.at[slot], sem_i[slot]).wait()

        offs_np = _offsets_np()

        def do_adds(slot, u):
            _, h, _ = decode(u)
            off_b = off_v[h, :]
            for j in range(_NG):
                for k in range(_GROW // _LANES):
                    sl = pl.ds(k * _LANES, _LANES)
                    idx_v[slot, j, sl] = idx_v[slot, j, sl] + off_b

        def fire_gathers(slot, u):
            _, h, _ = decode(u)
            off_s = jnp.int32(0)
            for k in range(1, 8):
                off_s = jnp.where(h >= k, jnp.int32(int(offs_np[k])), off_s)
            base = table_hbm.at[pl.ds(off_s, _PRIMES[0])]
            for j in range(_NG):
                pltpu.async_copy(
                    base.at[idx_v.at[slot, j]],
                    rows_v.at[slot, pl.ds(j * _GROW, _GROW)],
                    sem_g[slot],
                )

        def wait_gathers(slot):
            pltpu.make_async_copy(table_hbm.at[pl.ds(0, _BQ)],
                                  rows_v.at[slot], sem_g[slot]).wait()

        cols = [jnp.full((_LANES,), d, jnp.int32) for d in range(_EMBED_DIM)]

        def do_transpose(slot):
            rows = rows_v.at[slot]

            def tbody(c_p, _):
                base = c_p * _GROW
                for l0 in range(0, _GROW, _LANES):
                    row_idx = base + (l0 + iota16)
                    vs = [plsc.load_gather(rows, [row_idx, cols[d]])
                          for d in range(_EMBED_DIM)]
                    for d in range(_EMBED_DIM):
                        trans_v[slot, d // 8, c_p, d % 8,
                                pl.ds(l0, _LANES)] = vs[d]
                return _

            lax.fori_loop(0, _NG, tbody, 0)

        def fire_out(slot, u):
            s, h, c0 = decode(u)
            for r2 in range(2):
                pltpu.async_copy(
                    trans_v.at[slot, r2],
                    out_hbm.at[s, h, r2, pl.ds(c0, _NG)],
                    sem_o[slot],
                )

        def wait_out(slot):
            for r2 in range(2):
                pltpu.make_async_copy(trans_v.at[slot, r2],
                                      out_hbm.at[0, 0, 0, pl.ds(0, _NG)],
                                      sem_o[slot]).wait()

        for b in range(_NBUF):
            fire_idx(b, base_u + b)
        for b in range(_NBUF):
            wait_idx(b)
            fire_gathers(b, base_u + b)
        for b in range(_NBUF):
            wait_gathers(b)
            do_transpose(b)
            fire_out(b, base_u + b)
            fire_idx(b, jnp.minimum(base_u + _NBUF + b, n_units - 1))

        def loop_body(g, _):
            u0 = base_u + _NBUF * g
            for b in range(_NBUF):
                wait_idx(b)
                fire_gathers(b, u0 + b)
            for b in range(_NBUF):
                wait_gathers(b)
                wait_out(b)
                do_transpose(b)
                fire_out(b, u0 + b)
                fire_idx(b, jnp.minimum(u0 + _NBUF + b, n_units - 1))
            return _

        lax.fori_loop(1, units_per_w // _NBUF, loop_body, 0)

        for b in range(_NBUF):
            wait_idx(b)
            wait_out(b)

    return body(table, idx6, off16)


def kernel(table, hash_indices):
    bb, s_dim, h_dim = hash_indices.shape
    idx6 = (hash_indices.astype(jnp.int32)
            .transpose(1, 2, 0)
            .reshape(s_dim, h_dim, bb // 128, 128)
            .transpose(0, 2, 1, 3))
    off16 = jnp.asarray(
        np.repeat(_offsets_np()[:, None], _LANES, axis=1), dtype=jnp.int32)
    tp4 = (jnp.pad(table.T, ((0, 0), (0, _VPAD - table.shape[0])))
           .reshape(2, 8, _VPAD // 128, 128)
           .transpose(0, 2, 1, 3))
    tlin = _sc_table_rowmajor(tp4)
    n_units = s_dim * h_dim * (bb // _BQ)
    out6 = _sc_gather(tlin, idx6, off16, n_units // _NW)
    return (out6.transpose(3, 5, 0, 1, 2, 4)
            .reshape(bb, s_dim, h_dim, _EMBED_DIM))


# speedup vs baseline: 3.3833x; 1.2083x over previous
import functools

import jax
import jax.numpy as jnp
import numpy as np
from jax import lax
from jax.experimental import pallas as pl
from jax.experimental.pallas import tpu as pltpu
from jax.experimental.pallas import tpu_sc as plsc

_PRIMES = [99991, 100003, 100019, 100043, 100057, 100069, 100103, 100109]
_EMBED_DIM = 16

_NC = 2
_NS = 16
_NW = _NC * _NS
_LANES = 16

_BQ = 1024
_GROW = 128
_NG = _BQ // _GROW
_NBUF = 2


def _offsets_np():
    offs = [0]
    for p in _PRIMES[:-1]:
        offs.append(offs[-1] + p)
    return np.asarray(offs, dtype=np.int32)


_VPAD = 851968
_TCH = 1024


@jax.jit
def _sc_table_rowmajor(tp4):
    mesh = plsc.VectorSubcoreMesh(core_axis_name="c", subcore_axis_name="s")
    cpw = _VPAD // (_NW * _TCH)
    ccols = _TCH // 128

    @functools.partial(
        pl.kernel,
        mesh=mesh,
        out_type=jax.ShapeDtypeStruct((_VPAD, _EMBED_DIM), jnp.float32),
        compiler_params=pltpu.CompilerParams(
            use_tc_tiling_on_sc=False, needs_layout_passes=False),
        scratch_types=[
            pltpu.VMEM((_NBUF, 2, _TCH // 128, 8, 128), jnp.float32),
            pltpu.VMEM((_NBUF, _TCH, _EMBED_DIM), jnp.float32),
            pltpu.SemaphoreType.DMA,
            pltpu.SemaphoreType.DMA,
            pltpu.SemaphoreType.DMA,
            pltpu.SemaphoreType.DMA,
        ],
    )
    def body(tp4_hbm, tlin_hbm, din, dout, si0, si1, so0, so1):
        sem_i = (si0, si1)
        sem_o = (so0, so1)
        wid = lax.axis_index("s") * _NC + lax.axis_index("c")
        base_c = wid * cpw
        iota16 = lax.iota(jnp.int32, _LANES)
        cols = [jnp.full((_LANES,), d, jnp.int32) for d in range(_EMBED_DIM)]

        def fire_in(slot, c):
            pltpu.async_copy(tp4_hbm.at[:, pl.ds(c * ccols, ccols)],
                             din.at[slot], sem_i[slot])

        def wait_in(slot):
            pltpu.make_async_copy(tp4_hbm.at[:, pl.ds(0, ccols)],
                                  din.at[slot], sem_i[slot]).wait()

        def fire_out(slot, c):
            pltpu.async_copy(dout.at[slot],
                             tlin_hbm.at[pl.ds(c * _TCH, _TCH)], sem_o[slot])

        def wait_out(slot):
            pltpu.make_async_copy(dout.at[slot],
                                  tlin_hbm.at[pl.ds(0, _TCH)],
                                  sem_o[slot]).wait()

        def do_transpose(slot):
            dref = dout.at[slot]

            def g_body(g, _):
                c_rel = g >> 3
                l0 = (g & 7) * _LANES
                row_idx = c_rel * 128 + l0 + iota16
                vs = [din[slot, d // 8, c_rel, d % 8, pl.ds(l0, _LANES)]
                      for d in range(_EMBED_DIM)]
                for d in range(_EMBED_DIM):
                    plsc.store_scatter(dref, [row_idx, cols[d]], vs[d])
                return _

            lax.fori_loop(0, _TCH // _LANES, g_body, 0)

        for b in range(_NBUF):
            fire_in(b, base_c + b)
        for b in range(_NBUF):
            wait_in(b)
            do_transpose(b)
            fire_out(b, base_c + b)
            fire_in(b, jnp.minimum(base_c + _NBUF + b, _VPAD // _TCH - 1))

        def loop_body(g, _):
            c0 = base_c + _NBUF * g
            for b in range(_NBUF):
                wait_in(b)
                wait_out(b)
                do_transpose(b)
                fire_out(b, c0 + b)
                fire_in(b, jnp.minimum(c0 + _NBUF + b, _VPAD // _TCH - 1))
            return _

        lax.fori_loop(1, cpw // _NBUF, loop_body, 0)

        for b in range(_NBUF):
            wait_in(b)
            wait_out(b)

    return body(tp4)


@functools.partial(jax.jit, static_argnames=("units_per_w",))
def _sc_gather(table, idx6, off16, units_per_w):
    s_dim, c_dim, h_dim, l_dim = idx6.shape
    n_units = s_dim * h_dim * (c_dim * l_dim // _BQ)
    mesh = plsc.VectorSubcoreMesh(core_axis_name="c", subcore_axis_name="s")

    @functools.partial(
        pl.kernel,
        mesh=mesh,
        out_type=jax.ShapeDtypeStruct(
            (s_dim, h_dim, 2, c_dim, 8, l_dim), jnp.float32),
        compiler_params=pltpu.CompilerParams(
            use_tc_tiling_on_sc=False, needs_layout_passes=False),
        scratch_types=[
            pltpu.VMEM((3, _NG, _GROW), jnp.int32),
            pltpu.VMEM((3, _BQ, _EMBED_DIM), jnp.float32),
            pltpu.VMEM((_NBUF, 2, _NG, 8, _GROW), jnp.float32),
            pltpu.VMEM((8, _LANES), jnp.int32),
            pltpu.SemaphoreType.DMA,
            pltpu.SemaphoreType.DMA,
            pltpu.SemaphoreType.DMA,
            pltpu.SemaphoreType.DMA,
            pltpu.SemaphoreType.DMA,
            pltpu.SemaphoreType.DMA,
            pltpu.SemaphoreType.DMA,
            pltpu.SemaphoreType.DMA,
        ],
    )
    def body(table_hbm, idx_hbm, off_hbm, out_hbm, idx_v, rows_v, trans_v,
             off_v, si0, si1, si2, sg0, sg1, sg2, so0, so1):
        sem_i = (si0, si1, si2)
        sem_g = (sg0, sg1, sg2)
        sem_o = (so0, so1)
        wid = lax.axis_index("s") * _NC + lax.axis_index("c")
        base_u = wid * units_per_w
        pltpu.sync_copy(off_hbm, off_v)
        iota16 = lax.iota(jnp.int32, _LANES)

        def decode(u):
            pair = u >> 2
            q = u & 3
            return pair >> 3, pair & 7, q * _NG

        def fire_idx(slot, u):
            s, h, c0 = decode(u)
            pltpu.async_copy(idx_hbm.at[s, pl.ds(c0, _NG), h],
                             idx_v.at[slot], sem_i[slot])

        def wait_idx(slot):
            pltpu.make_async_copy(idx_hbm.at[0, pl.ds(0, _NG), 0],
                                  idx_v.at[slot], sem_i[slot]).wait()

        offs_np = _offsets_np()

        def do_adds(slot, u):
            _, h, _ = decode(u)
            off_b = off_v[h, :]
            for j in range(_NG):
                for k in range(_GROW // _LANES):
                    sl = pl.ds(k * _LANES, _LANES)
                    idx_v[slot, j, sl] = idx_v[slot, j, sl] + off_b

        def fire_gathers(slot, u):
            _, h, _ = decode(u)
            off_s = jnp.int32(0)
            for k in range(1, 8):
                off_s = jnp.where(h >= k, jnp.int32(int(offs_np[k])), off_s)
            base = table_hbm.at[pl.ds(off_s, _PRIMES[0])]
            for j in range(_NG):
                pltpu.async_copy(
                    base.at[idx_v.at[slot, j]],
                    rows_v.at[slot, pl.ds(j * _GROW, _GROW)],
                    sem_g[slot],
                )

        def wait_gathers(slot):
            pltpu.make_async_copy(table_hbm.at[pl.ds(0, _BQ)],
                                  rows_v.at[slot], sem_g[slot]).wait()

        cols = [jnp.full((_LANES,), d, jnp.int32) for d in range(_EMBED_DIM)]

        def do_transpose(rslot, tslot):
            rows = rows_v.at[rslot]

            def tbody(c_p, _):
                base = c_p * _GROW
                for l0 in range(0, _GROW, _LANES):
                    row_idx = base + (l0 + iota16)
                    vs = [plsc.load_gather(rows, [row_idx, cols[d]])
                          for d in range(_EMBED_DIM)]
                    for d in range(_EMBED_DIM):
                        trans_v[tslot, d // 8, c_p, d % 8,
                                pl.ds(l0, _LANES)] = vs[d]
                return _

            lax.fori_loop(0, _NG, tbody, 0)

        def fire_out(slot, u):
            s, h, c0 = decode(u)
            for r2 in range(2):
                pltpu.async_copy(
                    trans_v.at[slot, r2],
                    out_hbm.at[s, h, r2, pl.ds(c0, _NG)],
                    sem_o[slot],
                )

        def wait_out(slot):
            for r2 in range(2):
                pltpu.make_async_copy(trans_v.at[slot, r2],
                                      out_hbm.at[0, 0, 0, pl.ds(0, _NG)],
                                      sem_o[slot]).wait()

        def p1(rs, u):
            wait_idx(rs)
            fire_gathers(rs, u)

        def p2(rs, ts, u):
            wait_gathers(rs)
            wait_out(ts)
            do_transpose(rs, ts)
            fire_out(ts, u)
            fire_idx(rs, jnp.minimum(u + 3, n_units - 1))

        for k in range(3):
            fire_idx(k, base_u + k)
        fire_out(0, base_u + 0)
        fire_out(1, base_u + 1)
        p1(0, base_u + 0)
        p1(1, base_u + 1)

        def loop_body(g, _):
            u0 = base_u + 2 + 6 * g
            for k in range(6):
                u = u0 + k
                p2(k % 3, k % 2, u - 2)
                p1((2 + k) % 3, u)
            return _

        lax.fori_loop(0, (units_per_w - 2) // 6, loop_body, 0)

        p2(0, 0, base_u + units_per_w - 2)
        p2(1, 1, base_u + units_per_w - 1)
        for k in (2, 0, 1):
            wait_idx(k)
        for t in range(2):
            wait_out(t)

    return body(table, idx6, off16)


def kernel(table, hash_indices):
    bb, s_dim, h_dim = hash_indices.shape
    idx6 = (hash_indices.astype(jnp.int32)
            .transpose(1, 2, 0)
            .reshape(s_dim, h_dim, bb // 128, 128)
            .transpose(0, 2, 1, 3))
    off16 = jnp.asarray(
        np.repeat(_offsets_np()[:, None], _LANES, axis=1), dtype=jnp.int32)
    tp4 = (jnp.pad(table.T, ((0, 0), (0, _VPAD - table.shape[0])))
           .reshape(2, 8, _VPAD // 128, 128)
           .transpose(0, 2, 1, 3))
    tlin = _sc_table_rowmajor(tp4)
    n_units = s_dim * h_dim * (bb // _BQ)
    out6 = _sc_gather(tlin, idx6, off16, n_units // _NW)
    return (out6.transpose(3, 5, 0, 1, 2, 4)
            .reshape(bb, s_dim, h_dim, _EMBED_DIM))


# speedup vs baseline: 3.4150x; 1.0094x over previous
import functools

import jax
import jax.numpy as jnp
import numpy as np
from jax import lax
from jax.experimental import pallas as pl
from jax.experimental.pallas import tpu as pltpu
from jax.experimental.pallas import tpu_sc as plsc

_PRIMES = [99991, 100003, 100019, 100043, 100057, 100069, 100103, 100109]
_EMBED_DIM = 16

_NC = 2
_NS = 16
_NW = _NC * _NS
_LANES = 16

_BQ = 1024
_GROW = 128
_NG = _BQ // _GROW
_NBUF = 2


def _offsets_np():
    offs = [0]
    for p in _PRIMES[:-1]:
        offs.append(offs[-1] + p)
    return np.asarray(offs, dtype=np.int32)


_VPAD = 851968
_TCH = 1024


@jax.jit
def _sc_table_rowmajor(tp4):
    mesh = plsc.VectorSubcoreMesh(core_axis_name="c", subcore_axis_name="s")
    cpw = _VPAD // (_NW * _TCH)
    ccols = _TCH // 128

    @functools.partial(
        pl.kernel,
        mesh=mesh,
        out_type=jax.ShapeDtypeStruct((_VPAD, _EMBED_DIM), jnp.float32),
        compiler_params=pltpu.CompilerParams(
            use_tc_tiling_on_sc=False, needs_layout_passes=False),
        scratch_types=[
            pltpu.VMEM((3, 2, _TCH // 128, 8, 128), jnp.float32),
            pltpu.VMEM((_NBUF, _TCH, _EMBED_DIM), jnp.float32),
            pltpu.SemaphoreType.DMA,
            pltpu.SemaphoreType.DMA,
            pltpu.SemaphoreType.DMA,
            pltpu.SemaphoreType.DMA,
            pltpu.SemaphoreType.DMA,
        ],
    )
    def body(tp4_hbm, tlin_hbm, din, dout, si0, si1, si2, so0, so1):
        sem_i = (si0, si1, si2)
        sem_o = (so0, so1)
        wid = lax.axis_index("s") * _NC + lax.axis_index("c")
        base_c = wid * cpw
        iota16 = lax.iota(jnp.int32, _LANES)
        cols = [jnp.full((_LANES,), d, jnp.int32) for d in range(_EMBED_DIM)]

        def fire_in(slot, c):
            pltpu.async_copy(tp4_hbm.at[:, pl.ds(c * ccols, ccols)],
                             din.at[slot], sem_i[slot])

        def wait_in(slot):
            pltpu.make_async_copy(tp4_hbm.at[:, pl.ds(0, ccols)],
                                  din.at[slot], sem_i[slot]).wait()

        def fire_out(slot, c):
            pltpu.async_copy(dout.at[slot],
                             tlin_hbm.at[pl.ds(c * _TCH, _TCH)], sem_o[slot])

        def wait_out(slot):
            pltpu.make_async_copy(dout.at[slot],
                                  tlin_hbm.at[pl.ds(0, _TCH)],
                                  sem_o[slot]).wait()

        def do_transpose(rslot, tslot):
            dref = dout.at[tslot]

            def g_body(g, _):
                c_rel = g >> 3
                l0 = (g & 7) * _LANES
                row_idx = c_rel * 128 + l0 + iota16
                vs = [din[rslot, d // 8, c_rel, d % 8, pl.ds(l0, _LANES)]
                      for d in range(_EMBED_DIM)]
                for d in range(_EMBED_DIM):
                    plsc.store_scatter(dref, [row_idx, cols[d]], vs[d])
                return _

            lax.fori_loop(0, _TCH // _LANES, g_body, 0)

        def step(rs, ts, c):
            wait_in(rs)
            wait_out(ts)
            do_transpose(rs, ts)
            fire_out(ts, c)
            fire_in(rs, jnp.minimum(c + 3, _VPAD // _TCH - 1))

        for k in range(3):
            fire_in(k, base_c + k)
        fire_out(0, base_c + 0)
        fire_out(1, base_c + 1)

        def loop_body(g, _):
            c0 = base_c + 6 * g
            for k in range(6):
                step(k % 3, k % 2, c0 + k)
            return _

        lax.fori_loop(0, (cpw - 2) // 6, loop_body, 0)

        step(0, 0, base_c + cpw - 2)
        step(1, 1, base_c + cpw - 1)
        for k in (2, 0, 1):
            wait_in(k)
        for t in range(2):
            wait_out(t)

    return body(tp4)


@functools.partial(jax.jit, static_argnames=("units_per_w",))
def _sc_gather(table, idx6, off16, units_per_w):
    s_dim, c_dim, h_dim, l_dim = idx6.shape
    n_units = s_dim * h_dim * (c_dim * l_dim // _BQ)
    mesh = plsc.VectorSubcoreMesh(core_axis_name="c", subcore_axis_name="s")

    @functools.partial(
        pl.kernel,
        mesh=mesh,
        out_type=jax.ShapeDtypeStruct(
            (s_dim, h_dim, 2, c_dim, 8, l_dim), jnp.float32),
        compiler_params=pltpu.CompilerParams(
            use_tc_tiling_on_sc=False, needs_layout_passes=False),
        scratch_types=[
            pltpu.VMEM((3, _NG, _GROW), jnp.int32),
            pltpu.VMEM((3, _BQ, _EMBED_DIM), jnp.float32),
            pltpu.VMEM((_NBUF, 2, _NG, 8, _GROW), jnp.float32),
            pltpu.VMEM((8, _LANES), jnp.int32),
            pltpu.SemaphoreType.DMA,
            pltpu.SemaphoreType.DMA,
            pltpu.SemaphoreType.DMA,
            pltpu.SemaphoreType.DMA,
            pltpu.SemaphoreType.DMA,
            pltpu.SemaphoreType.DMA,
            pltpu.SemaphoreType.DMA,
            pltpu.SemaphoreType.DMA,
        ],
    )
    def body(table_hbm, idx_hbm, off_hbm, out_hbm, idx_v, rows_v, trans_v,
             off_v, si0, si1, si2, sg0, sg1, sg2, so0, so1):
        sem_i = (si0, si1, si2)
        sem_g = (sg0, sg1, sg2)
        sem_o = (so0, so1)
        wid = lax.axis_index("s") * _NC + lax.axis_index("c")
        base_u = wid * units_per_w
        pltpu.sync_copy(off_hbm, off_v)
        iota16 = lax.iota(jnp.int32, _LANES)

        def decode(u):
            pair = u >> 2
            q = u & 3
            return pair >> 3, pair & 7, q * _NG

        def fire_idx(slot, u):
            s, h, c0 = decode(u)
            pltpu.async_copy(idx_hbm.at[s, pl.ds(c0, _NG), h],
                             idx_v.at[slot], sem_i[slot])

        def wait_idx(slot):
            pltpu.make_async_copy(idx_hbm.at[0, pl.ds(0, _NG), 0],
                                  idx_v.at[slot], sem_i[slot]).wait()

        offs_np = _offsets_np()

        def do_adds(slot, u):
            _, h, _ = decode(u)
            off_b = off_v[h, :]
            for j in range(_NG):
                for k in range(_GROW // _LANES):
                    sl = pl.ds(k * _LANES, _LANES)
                    idx_v[slot, j, sl] = idx_v[slot, j, sl] + off_b

        def fire_gathers(slot, u):
            _, h, _ = decode(u)
            off_s = jnp.int32(0)
            for k in range(1, 8):
                off_s = jnp.where(h >= k, jnp.int32(int(offs_np[k])), off_s)
            base = table_hbm.at[pl.ds(off_s, _PRIMES[0])]
            for j in range(_NG):
                pltpu.async_copy(
                    base.at[idx_v.at[slot, j]],
                    rows_v.at[slot, pl.ds(j * _GROW, _GROW)],
                    sem_g[slot],
                )

        def wait_gathers(slot):
            pltpu.make_async_copy(table_hbm.at[pl.ds(0, _BQ)],
                                  rows_v.at[slot], sem_g[slot]).wait()

        cols = [jnp.full((_LANES,), d, jnp.int32) for d in range(_EMBED_DIM)]

        def do_transpose(rslot, tslot):
            rows = rows_v.at[rslot]

            def tbody(c_p, _):
                base = c_p * _GROW
                for l0 in range(0, _GROW, _LANES):
                    row_idx = base + (l0 + iota16)
                    vs = [plsc.load_gather(rows, [row_idx, cols[d]])
                          for d in range(_EMBED_DIM)]
                    for d in range(_EMBED_DIM):
                        trans_v[tslot, d // 8, c_p, d % 8,
                                pl.ds(l0, _LANES)] = vs[d]
                return _

            lax.fori_loop(0, _NG, tbody, 0)

        def fire_out(slot, u):
            s, h, c0 = decode(u)
            for r2 in range(2):
                pltpu.async_copy(
                    trans_v.at[slot, r2],
                    out_hbm.at[s, h, r2, pl.ds(c0, _NG)],
                    sem_o[slot],
                )

        def wait_out(slot):
            for r2 in range(2):
                pltpu.make_async_copy(trans_v.at[slot, r2],
                                      out_hbm.at[0, 0, 0, pl.ds(0, _NG)],
                                      sem_o[slot]).wait()

        def p1(rs, u):
            wait_idx(rs)
            fire_gathers(rs, u)

        def p2(rs, ts, u):
            wait_gathers(rs)
            wait_out(ts)
            do_transpose(rs, ts)
            fire_out(ts, u)
            fire_idx(rs, jnp.minimum(u + 3, n_units - 1))

        for k in range(3):
            fire_idx(k, base_u + k)
        fire_out(0, base_u + 0)
        fire_out(1, base_u + 1)
        p1(0, base_u + 0)
        p1(1, base_u + 1)

        def loop_body(g, _):
            u0 = base_u + 2 + 6 * g
            for k in range(6):
                u = u0 + k
                p2(k % 3, k % 2, u - 2)
                p1((2 + k) % 3, u)
            return _

        lax.fori_loop(0, (units_per_w - 2) // 6, loop_body, 0)

        p2(0, 0, base_u + units_per_w - 2)
        p2(1, 1, base_u + units_per_w - 1)
        for k in (2, 0, 1):
            wait_idx(k)
        for t in range(2):
            wait_out(t)

    return body(table, idx6, off16)


def kernel(table, hash_indices):
    bb, s_dim, h_dim = hash_indices.shape
    idx6 = (hash_indices.astype(jnp.int32)
            .transpose(1, 2, 0)
            .reshape(s_dim, h_dim, bb // 128, 128)
            .transpose(0, 2, 1, 3))
    off16 = jnp.asarray(
        np.repeat(_offsets_np()[:, None], _LANES, axis=1), dtype=jnp.int32)
    tp4 = (jnp.pad(table.T, ((0, 0), (0, _VPAD - table.shape[0])))
           .reshape(2, 8, _VPAD // 128, 128)
           .transpose(0, 2, 1, 3))
    tlin = _sc_table_rowmajor(tp4)
    n_units = s_dim * h_dim * (bb // _BQ)
    out6 = _sc_gather(tlin, idx6, off16, n_units // _NW)
    return (out6.transpose(3, 5, 0, 1, 2, 4)
            .reshape(bb, s_dim, h_dim, _EMBED_DIM))


# speedup vs baseline: 3.4262x; 1.0033x over previous
import functools

import jax
import jax.numpy as jnp
import numpy as np
from jax import lax
from jax.experimental import pallas as pl
from jax.experimental.pallas import tpu as pltpu
from jax.experimental.pallas import tpu_sc as plsc

_PRIMES = [99991, 100003, 100019, 100043, 100057, 100069, 100103, 100109]
_EMBED_DIM = 16

_NC = 2
_NS = 16
_NW = _NC * _NS
_LANES = 16

_BQ = 1024
_GROW = 128
_NG = _BQ // _GROW
_NBUF = 2


def _offsets_np():
    offs = [0]
    for p in _PRIMES[:-1]:
        offs.append(offs[-1] + p)
    return np.asarray(offs, dtype=np.int32)


_VPAD = 851968
_TCH = 1024


@jax.jit
def _sc_table_rowmajor(tp4):
    mesh = plsc.VectorSubcoreMesh(core_axis_name="c", subcore_axis_name="s")
    cpw = _VPAD // (_NW * _TCH)
    ccols = _TCH // 128

    @functools.partial(
        pl.kernel,
        mesh=mesh,
        out_type=jax.ShapeDtypeStruct((_VPAD, _EMBED_DIM), jnp.float32),
        compiler_params=pltpu.CompilerParams(
            use_tc_tiling_on_sc=False, needs_layout_passes=False),
        scratch_types=[
            pltpu.VMEM((3, 2, _TCH // 128, 8, 128), jnp.float32),
            pltpu.VMEM((_NBUF, _TCH, _EMBED_DIM), jnp.float32),
            pltpu.SemaphoreType.DMA,
            pltpu.SemaphoreType.DMA,
            pltpu.SemaphoreType.DMA,
            pltpu.SemaphoreType.DMA,
            pltpu.SemaphoreType.DMA,
        ],
    )
    def body(tp4_hbm, tlin_hbm, din, dout, si0, si1, si2, so0, so1):
        sem_i = (si0, si1, si2)
        sem_o = (so0, so1)
        wid = lax.axis_index("s") * _NC + lax.axis_index("c")
        base_c = wid * cpw
        iota16 = lax.iota(jnp.int32, _LANES)
        cols = [jnp.full((_LANES,), d, jnp.int32) for d in range(_EMBED_DIM)]

        def fire_in(slot, c):
            pltpu.async_copy(tp4_hbm.at[:, pl.ds(c * ccols, ccols)],
                             din.at[slot], sem_i[slot])

        def wait_in(slot):
            pltpu.make_async_copy(tp4_hbm.at[:, pl.ds(0, ccols)],
                                  din.at[slot], sem_i[slot]).wait()

        def fire_out(slot, c):
            pltpu.async_copy(dout.at[slot],
                             tlin_hbm.at[pl.ds(c * _TCH, _TCH)], sem_o[slot])

        def wait_out(slot):
            pltpu.make_async_copy(dout.at[slot],
                                  tlin_hbm.at[pl.ds(0, _TCH)],
                                  sem_o[slot]).wait()

        def do_transpose(rslot, tslot):
            dref = dout.at[tslot]

            def g_body(g, _):
                c_rel = g >> 3
                l0 = (g & 7) * _LANES
                row_idx = c_rel * 128 + l0 + iota16
                vs = [din[rslot, d // 8, c_rel, d % 8, pl.ds(l0, _LANES)]
                      for d in range(_EMBED_DIM)]
                for d in range(_EMBED_DIM):
                    plsc.store_scatter(dref, [row_idx, cols[d]], vs[d])
                return _

            lax.fori_loop(0, _TCH // _LANES, g_body, 0)

        def step(rs, ts, c):
            wait_in(rs)
            wait_out(ts)
            do_transpose(rs, ts)
            fire_out(ts, c)
            fire_in(rs, jnp.minimum(c + 3, _VPAD // _TCH - 1))

        for k in range(3):
            fire_in(k, base_c + k)
        fire_out(0, base_c + 0)
        fire_out(1, base_c + 1)

        def loop_body(g, _):
            c0 = base_c + 6 * g
            for k in range(6):
                step(k % 3, k % 2, c0 + k)
            return _

        lax.fori_loop(0, (cpw - 2) // 6, loop_body, 0)

        step(0, 0, base_c + cpw - 2)
        step(1, 1, base_c + cpw - 1)
        for k in (2, 0, 1):
            wait_in(k)
        for t in range(2):
            wait_out(t)

    return body(tp4)


@functools.partial(jax.jit, static_argnames=("units_per_w",))
def _sc_gather(table, idx6, units_per_w):
    s_dim, c_dim, h_dim, l_dim = idx6.shape
    n_units = s_dim * h_dim * (c_dim * l_dim // _BQ)
    mesh = plsc.VectorSubcoreMesh(core_axis_name="c", subcore_axis_name="s")

    @functools.partial(
        pl.kernel,
        mesh=mesh,
        out_type=jax.ShapeDtypeStruct(
            (s_dim, h_dim, 2, c_dim, 8, l_dim), jnp.float32),
        compiler_params=pltpu.CompilerParams(
            use_tc_tiling_on_sc=False, needs_layout_passes=False),
        scratch_types=[
            pltpu.VMEM((3, _NG, _GROW), jnp.int32),
            pltpu.VMEM((3, _BQ, _EMBED_DIM), jnp.float32),
            pltpu.VMEM((_NBUF, 2, _NG, 8, _GROW), jnp.float32),
            pltpu.SemaphoreType.DMA,
            pltpu.SemaphoreType.DMA,
            pltpu.SemaphoreType.DMA,
            pltpu.SemaphoreType.DMA,
            pltpu.SemaphoreType.DMA,
            pltpu.SemaphoreType.DMA,
            pltpu.SemaphoreType.DMA,
            pltpu.SemaphoreType.DMA,
        ],
    )
    def body(table_hbm, idx_hbm, out_hbm, idx_v, rows_v, trans_v,
             si0, si1, si2, sg0, sg1, sg2, so0, so1):
        sem_i = (si0, si1, si2)
        sem_g = (sg0, sg1, sg2)
        sem_o = (so0, so1)
        wid = lax.axis_index("s") * _NC + lax.axis_index("c")
        base_u = wid * units_per_w
        iota16 = lax.iota(jnp.int32, _LANES)

        def decode(u):
            pair = u >> 2
            q = u & 3
            return pair >> 3, pair & 7, q * _NG

        def fire_idx(slot, u):
            s, h, c0 = decode(u)
            pltpu.async_copy(idx_hbm.at[s, pl.ds(c0, _NG), h],
                             idx_v.at[slot], sem_i[slot])

        def wait_idx(slot):
            pltpu.make_async_copy(idx_hbm.at[0, pl.ds(0, _NG), 0],
                                  idx_v.at[slot], sem_i[slot]).wait()

        offs_np = _offsets_np()

        def fire_gathers(slot, u):
            _, h, _ = decode(u)
            off_s = jnp.int32(0)
            for k in range(1, 8):
                off_s = jnp.where(h >= k, jnp.int32(int(offs_np[k])), off_s)
            base = table_hbm.at[pl.ds(off_s, _PRIMES[0])]
            for j in range(_NG):
                pltpu.async_copy(
                    base.at[idx_v.at[slot, j]],
                    rows_v.at[slot, pl.ds(j * _GROW, _GROW)],
                    sem_g[slot],
                )

        def wait_gathers(slot):
            pltpu.make_async_copy(table_hbm.at[pl.ds(0, _BQ)],
                                  rows_v.at[slot], sem_g[slot]).wait()

        cols = [jnp.full((_LANES,), d, jnp.int32) for d in range(_EMBED_DIM)]

        def do_transpose(rslot, tslot):
            rows = rows_v.at[rslot]

            def tbody(c_p, _):
                base = c_p * _GROW
                for l0 in range(0, _GROW, _LANES):
                    row_idx = base + (l0 + iota16)
                    vs = [plsc.load_gather(rows, [row_idx, cols[d]])
                          for d in range(_EMBED_DIM)]
                    for d in range(_EMBED_DIM):
                        trans_v[tslot, d // 8, c_p, d % 8,
                                pl.ds(l0, _LANES)] = vs[d]
                return _

            lax.fori_loop(0, _NG, tbody, 0)

        def fire_out(slot, u):
            s, h, c0 = decode(u)
            for r2 in range(2):
                pltpu.async_copy(
                    trans_v.at[slot, r2],
                    out_hbm.at[s, h, r2, pl.ds(c0, _NG)],
                    sem_o[slot],
                )

        def wait_out(slot):
            for r2 in range(2):
                pltpu.make_async_copy(trans_v.at[slot, r2],
                                      out_hbm.at[0, 0, 0, pl.ds(0, _NG)],
                                      sem_o[slot]).wait()

        def p1(rs, u):
            wait_idx(rs)
            fire_gathers(rs, u)

        def p2(rs, ts, u):
            wait_gathers(rs)
            wait_out(ts)
            do_transpose(rs, ts)
            fire_out(ts, u)
            fire_idx(rs, jnp.minimum(u + 3, n_units - 1))

        for k in range(3):
            fire_idx(k, base_u + k)
        fire_out(0, base_u + 0)
        fire_out(1, base_u + 1)
        p1(0, base_u + 0)
        p1(1, base_u + 1)

        def loop_body(g, _):
            u0 = base_u + 2 + 6 * g
            for k in range(6):
                u = u0 + k
                p2(k % 3, k % 2, u - 2)
                p1((2 + k) % 3, u)
            return _

        lax.fori_loop(0, (units_per_w - 2) // 6, loop_body, 0)

        p2(0, 0, base_u + units_per_w - 2)
        p2(1, 1, base_u + units_per_w - 1)
        for k in (2, 0, 1):
            wait_idx(k)
        for t in range(2):
            wait_out(t)

    return body(table, idx6)


def kernel(table, hash_indices):
    bb, s_dim, h_dim = hash_indices.shape
    idx6 = (hash_indices.astype(jnp.int32)
            .transpose(1, 2, 0)
            .reshape(s_dim, h_dim, bb // 128, 128)
            .transpose(0, 2, 1, 3))
    tp4 = (jnp.pad(table.T, ((0, 0), (0, _VPAD - table.shape[0])))
           .reshape(2, 8, _VPAD // 128, 128)
           .transpose(0, 2, 1, 3))
    tlin = _sc_table_rowmajor(tp4)
    n_units = s_dim * h_dim * (bb // _BQ)
    out6 = _sc_gather(tlin, idx6, n_units // _NW)
    return (out6.transpose(3, 5, 0, 1, 2, 4)
            .reshape(bb, s_dim, h_dim, _EMBED_DIM))
